# baseline (speedup 1.0000x reference)
import functools

import jax
import jax.numpy as jnp
from jax import lax
from jax.experimental import pallas as pl
from jax.experimental.pallas import tpu as pltpu
from jax.experimental.pallas import tpu_sc as plsc

N_NODES = 10000
D = 128
R = 8
N_EDGES = 160000

NC = 2
NS = 16
BATCH = 128
NB = 80
NCHUNK = 2
CB = NB // NCHUNK
NBT = NS * NB
E_PAD = NBT * BATCH
N_PAD = 10112
SLAB = N_PAD // NS
NBUF = 2


def _relmm_body(f_ref, w_ref, o_ref):
    o_ref[...] = jnp.dot(f_ref[...], w_ref[...],
                         preferred_element_type=jnp.float32)


def _rel_matmul(feat, w_cat):
    bn = 1000
    return pl.pallas_call(
        _relmm_body,
        grid=(N_NODES // bn,),
        in_specs=[
            pl.BlockSpec((bn, D), lambda n: (n, 0)),
            pl.BlockSpec((D, R * D), lambda n: (0, 0)),
        ],
        out_specs=pl.BlockSpec((bn, R * D), lambda n: (n, 0)),
        out_shape=jax.ShapeDtypeStruct((N_NODES, R * D), jnp.float32),
    )(feat, w_cat)


def _sc_gather_scatter(f_table, src_w, et_w, dst_w):
    mesh = plsc.VectorSubcoreMesh(core_axis_name="c", subcore_axis_name="s")

    @functools.partial(
        pl.kernel,
        mesh=mesh,
        out_type=jax.ShapeDtypeStruct((N_PAD, D), jnp.float32),
        scratch_types=[
            pltpu.VMEM((CB, BATCH), jnp.int32),
            pltpu.VMEM((CB, BATCH), jnp.int32),
            pltpu.VMEM((BATCH, D), jnp.float32),
            pltpu.VMEM((BATCH, D), jnp.float32),
            pltpu.VMEM_SHARED((N_PAD, D), jnp.float32),
            pltpu.SemaphoreType.DMA,
            pltpu.SemaphoreType.DMA,
        ],
    )
    def sc_kern(f_hbm, src_hbm, et_hbm, dst_hbm, out_hbm,
                key_v, dst_v, rows0, rows1, acc, sem0, sem1):
        cid = lax.axis_index("c")
        sid = lax.axis_index("s")

        @pl.when(cid == 0)
        def _all_work():
            with jax.named_scope("ph_zero"):
                zero16 = jnp.zeros((16,), jnp.float32)

                def zero_body(i, _):
                    for c in range(D // 16):
                        rows0[i, pl.ds(c * 16, 16)] = zero16
                    return _

                lax.fori_loop(0, BATCH, zero_body, None)
                for k in range(SLAB // BATCH):
                    pltpu.sync_copy(
                        rows0, acc.at[pl.ds(sid * SLAB + k * BATCH, BATCH)])
                rem = SLAB % BATCH
                pltpu.sync_copy(
                    rows0.at[pl.ds(0, rem)],
                    acc.at[pl.ds(sid * SLAB + (SLAB // BATCH) * BATCH, rem)])

            plsc.subcore_barrier()

            bufs = (rows0, rows1)
            sems = (sem0, sem1)

            def gstart(bidx, b):
                pltpu.async_copy(f_hbm.at[key_v.at[bidx]], bufs[b], sems[b])

            for ci in range(NCHUNK):
                bofs = sid * NB + ci * CB

                with jax.named_scope("ph_meta"):
                    pltpu.sync_copy(src_hbm.at[pl.ds(bofs, CB)],
                                    key_v.at[pl.ds(0, CB)])
                    pltpu.sync_copy(et_hbm.at[pl.ds(bofs, CB)],
                                    dst_v.at[pl.ds(0, CB)])

                with jax.named_scope("ph_keys"):
                    def key_body(j, _):
                        for c in range(BATCH // 16):
                            sl = pl.ds(c * 16, 16)
                            key_v[j, sl] = key_v[j, sl] * R + dst_v[j, sl]
                        return _

                    lax.fori_loop(0, CB, key_body, None)

                with jax.named_scope("ph_dst"):
                    pltpu.sync_copy(dst_hbm.at[pl.ds(bofs, CB)],
                                    dst_v.at[pl.ds(0, CB)])

                def pipe_body(j, _):
                    for b in range(NBUF):
                        idx = j * NBUF + b
                        pltpu.make_async_copy(
                            f_hbm.at[pl.ds(0, BATCH)], bufs[b], sems[b]
                        ).wait()
                        pltpu.sync_copy(bufs[b], acc.at[dst_v.at[idx]],
                                        add=True)

                        @pl.when(idx + NBUF < CB)
                        def _start_next():
                            gstart(idx + NBUF, b)
                    return _

                with jax.named_scope("ph_pipe"):
                    for b in range(NBUF):
                        gstart(b, b)
                    lax.fori_loop(0, CB // NBUF, pipe_body, None)

            plsc.subcore_barrier()
            with jax.named_scope("ph_out"):
                pltpu.sync_copy(acc.at[pl.ds(sid * SLAB, SLAB)],
                                out_hbm.at[pl.ds(sid * SLAB, SLAB)])

    return sc_kern(f_table, src_w, et_w, dst_w)


def kernel(feat, edge_index, etypes, weight):
    src = edge_index[0]
    dst = edge_index[1]
    pad = E_PAD - N_EDGES
    src_p = jnp.concatenate([src, jnp.zeros((pad,), jnp.int32)])
    et_p = jnp.concatenate([etypes, jnp.zeros((pad,), jnp.int32)])
    dst_p = jnp.concatenate(
        [dst, N_NODES + (jnp.arange(pad, dtype=jnp.int32) % (N_PAD - N_NODES))]
    )

    w_cat = jnp.transpose(weight, (1, 0, 2)).reshape(D, R * D)
    f_table = _rel_matmul(feat, w_cat).reshape(N_NODES * R, D)
    out = _sc_gather_scatter(
        f_table,
        src_p.reshape(NBT, BATCH),
        et_p.reshape(NBT, BATCH),
        dst_p.reshape(NBT, BATCH),
    )
    return out[:N_NODES]


# speedup vs baseline: 2.3545x; 2.3545x over previous
import functools

import jax
import jax.numpy as jnp
from jax import lax
from jax.experimental import pallas as pl
from jax.experimental.pallas import tpu as pltpu
from jax.experimental.pallas import tpu_sc as plsc

N_NODES = 10000
D = 128
R = 8
N_EDGES = 160000

NC = 2
NS = 16
BATCH = 128
NB = 80
NCHUNK = 2
CB = NB // NCHUNK
NBT = NS * NB
E_PAD = NBT * BATCH
N_PAD = 10112
SLAB = N_PAD // NS
NBUF = 2


def _relmm_body(f_ref, w_ref, o_ref):
    o_ref[0] = jnp.dot(f_ref[...], w_ref[0],
                       preferred_element_type=jnp.float32)


def _rel_matmul(feat, weight):
    return pl.pallas_call(
        _relmm_body,
        grid=(R,),
        in_specs=[
            pl.BlockSpec((N_NODES, D), lambda r: (0, 0)),
            pl.BlockSpec((1, D, D), lambda r: (r, 0, 0)),
        ],
        out_specs=pl.BlockSpec((1, N_NODES, D), lambda r: (r, 0, 0)),
        out_shape=jax.ShapeDtypeStruct((R, N_NODES, D), jnp.float32),
    )(feat, weight)


def _sc_gather_scatter(f_table, src_w, et_w, dst_w):
    mesh = plsc.VectorSubcoreMesh(core_axis_name="c", subcore_axis_name="s")

    @functools.partial(
        pl.kernel,
        mesh=mesh,
        out_type=jax.ShapeDtypeStruct((N_PAD, D), jnp.float32),
        scratch_types=[
            pltpu.VMEM((CB, BATCH), jnp.int32),
            pltpu.VMEM((CB, BATCH), jnp.int32),
            pltpu.VMEM((BATCH, D), jnp.float32),
            pltpu.VMEM((BATCH, D), jnp.float32),
            pltpu.VMEM_SHARED((N_PAD, D), jnp.float32),
            pltpu.SemaphoreType.DMA,
            pltpu.SemaphoreType.DMA,
        ],
    )
    def sc_kern(f_hbm, src_hbm, et_hbm, dst_hbm, out_hbm,
                key_v, dst_v, rows0, rows1, acc, sem0, sem1):
        cid = lax.axis_index("c")
        sid = lax.axis_index("s")

        @pl.when(cid == 0)
        def _all_work():
            with jax.named_scope("ph_zero"):
                zero16 = jnp.zeros((16,), jnp.float32)

                def zero_body(i, _):
                    for c in range(D // 16):
                        rows0[i, pl.ds(c * 16, 16)] = zero16
                    return _

                lax.fori_loop(0, BATCH, zero_body, None)
                for k in range(SLAB // BATCH):
                    pltpu.sync_copy(
                        rows0, acc.at[pl.ds(sid * SLAB + k * BATCH, BATCH)])
                rem = SLAB % BATCH
                pltpu.sync_copy(
                    rows0.at[pl.ds(0, rem)],
                    acc.at[pl.ds(sid * SLAB + (SLAB // BATCH) * BATCH, rem)])

            plsc.subcore_barrier()

            bufs = (rows0, rows1)
            sems = (sem0, sem1)

            def gstart(bidx, b):
                pltpu.async_copy(f_hbm.at[key_v.at[bidx]], bufs[b], sems[b])

            for ci in range(NCHUNK):
                bofs = sid * NB + ci * CB

                with jax.named_scope("ph_meta"):
                    pltpu.sync_copy(et_hbm.at[pl.ds(bofs, CB)],
                                    key_v.at[pl.ds(0, CB)])
                    pltpu.sync_copy(src_hbm.at[pl.ds(bofs, CB)],
                                    dst_v.at[pl.ds(0, CB)])

                with jax.named_scope("ph_keys"):
                    def key_body(j, _):
                        for c in range(BATCH // 16):
                            sl = pl.ds(c * 16, 16)
                            key_v[j, sl] = key_v[j, sl] * N_NODES + dst_v[j, sl]
                        return _

                    lax.fori_loop(0, CB, key_body, None)

                with jax.named_scope("ph_dst"):
                    pltpu.sync_copy(dst_hbm.at[pl.ds(bofs, CB)],
                                    dst_v.at[pl.ds(0, CB)])

                def pipe_body(j, _):
                    for b in range(NBUF):
                        idx = j * NBUF + b
                        pltpu.make_async_copy(
                            f_hbm.at[pl.ds(0, BATCH)], bufs[b], sems[b]
                        ).wait()
                        pltpu.sync_copy(bufs[b], acc.at[dst_v.at[idx]],
                                        add=True)

                        @pl.when(idx + NBUF < CB)
                        def _start_next():
                            gstart(idx + NBUF, b)
                    return _

                with jax.named_scope("ph_pipe"):
                    for b in range(NBUF):
                        gstart(b, b)
                    lax.fori_loop(0, CB // NBUF, pipe_body, None)

            plsc.subcore_barrier()
            with jax.named_scope("ph_out"):
                pltpu.sync_copy(acc.at[pl.ds(sid * SLAB, SLAB)],
                                out_hbm.at[pl.ds(sid * SLAB, SLAB)])

    return sc_kern(f_table, src_w, et_w, dst_w)


def kernel(feat, edge_index, etypes, weight):
    src = edge_index[0]
    dst = edge_index[1]
    pad = E_PAD - N_EDGES
    ar = jnp.arange(pad, dtype=jnp.int32)
    src_p = jnp.concatenate([src, (ar * 37) % N_NODES])
    et_p = jnp.concatenate([etypes, ar % R])
    dst_p = jnp.concatenate([dst, N_NODES + ar % (N_PAD - N_NODES)])

    f_table = _rel_matmul(feat, weight).reshape(N_NODES * R, D)
    out = _sc_gather_scatter(
        f_table,
        src_p.reshape(NBT, BATCH),
        et_p.reshape(NBT, BATCH),
        dst_p.reshape(NBT, BATCH),
    )
    return out[:N_NODES]


# speedup vs baseline: 2.4607x; 1.0451x over previous
import functools

import jax
import jax.numpy as jnp
from jax import lax
from jax.experimental import pallas as pl
from jax.experimental.pallas import tpu as pltpu
from jax.experimental.pallas import tpu_sc as plsc

N_NODES = 10000
D = 128
R = 8
N_EDGES = 160000

NC = 2
NS = 16
BATCH = 64
NB = 160
NCHUNK = 4
CB = NB // NCHUNK
NBT = NS * NB
E_PAD = NBT * BATCH
N_PAD = 10112
SLAB = N_PAD // NS
SLAB_LAST = N_NODES - 15 * SLAB
NBUF = 4


def _relmm_body(f_ref, w_ref, o_ref):
    o_ref[0] = jnp.dot(f_ref[...], w_ref[0],
                       preferred_element_type=jnp.float32)


def _rel_matmul(feat, weight):
    return pl.pallas_call(
        _relmm_body,
        grid=(R,),
        in_specs=[
            pl.BlockSpec((N_NODES, D), lambda r: (0, 0)),
            pl.BlockSpec((1, D, D), lambda r: (r, 0, 0)),
        ],
        out_specs=pl.BlockSpec((1, N_NODES, D), lambda r: (r, 0, 0)),
        out_shape=jax.ShapeDtypeStruct((R, N_NODES, D), jnp.float32),
    )(feat, weight)


def _sc_gather_scatter(f_table, src_w, et_w, dst_w):
    mesh = plsc.VectorSubcoreMesh(core_axis_name="c", subcore_axis_name="s")

    @functools.partial(
        pl.kernel,
        mesh=mesh,
        out_type=jax.ShapeDtypeStruct((N_NODES, D), jnp.float32),
        scratch_types=[
            pltpu.VMEM((CB, BATCH), jnp.int32),
            pltpu.VMEM((CB, BATCH), jnp.int32),
            pltpu.VMEM((BATCH, D), jnp.float32),
            pltpu.VMEM((BATCH, D), jnp.float32),
            pltpu.VMEM((BATCH, D), jnp.float32),
            pltpu.VMEM((BATCH, D), jnp.float32),
            pltpu.VMEM_SHARED((N_PAD, D), jnp.float32),
            pltpu.SemaphoreType.DMA,
            pltpu.SemaphoreType.DMA,
            pltpu.SemaphoreType.DMA,
            pltpu.SemaphoreType.DMA,
        ],
    )
    def sc_kern(f_hbm, src_hbm, et_hbm, dst_hbm, out_hbm,
                key_v, dst_v, rows0, rows1, rows2, rows3, acc,
                sem0, sem1, sem2, sem3):
        cid = lax.axis_index("c")
        sid = lax.axis_index("s")

        @pl.when(cid == 0)
        def _all_work():
            with jax.named_scope("ph_zero"):
                zero16 = jnp.zeros((16,), jnp.float32)

                def zero_body(i, _):
                    for c in range(D // 16):
                        rows0[i, pl.ds(c * 16, 16)] = zero16
                    return _

                lax.fori_loop(0, BATCH, zero_body, None)
                for k in range(SLAB // BATCH):
                    pltpu.sync_copy(
                        rows0, acc.at[pl.ds(sid * SLAB + k * BATCH, BATCH)])
                rem = SLAB % BATCH
                if rem:
                    pltpu.sync_copy(
                        rows0.at[pl.ds(0, rem)],
                        acc.at[pl.ds(sid * SLAB + (SLAB // BATCH) * BATCH,
                                     rem)])

            plsc.subcore_barrier()

            bufs = (rows0, rows1, rows2, rows3)
            sems = (sem0, sem1, sem2, sem3)

            def gstart(bidx, b):
                pltpu.async_copy(f_hbm.at[key_v.at[bidx]], bufs[b], sems[b])

            for ci in range(NCHUNK):
                bofs = sid * NB + ci * CB

                with jax.named_scope("ph_meta"):
                    pltpu.sync_copy(et_hbm.at[pl.ds(bofs, CB)],
                                    key_v.at[pl.ds(0, CB)])
                    pltpu.sync_copy(src_hbm.at[pl.ds(bofs, CB)],
                                    dst_v.at[pl.ds(0, CB)])

                with jax.named_scope("ph_keys"):
                    def key_body(j, _):
                        for c in range(BATCH // 16):
                            sl = pl.ds(c * 16, 16)
                            key_v[j, sl] = key_v[j, sl] * N_NODES + dst_v[j, sl]
                        return _

                    lax.fori_loop(0, CB, key_body, None)

                with jax.named_scope("ph_dst"):
                    pltpu.sync_copy(dst_hbm.at[pl.ds(bofs, CB)],
                                    dst_v.at[pl.ds(0, CB)])

                def pipe_body(j, _):
                    for b in range(NBUF):
                        idx = j * NBUF + b
                        pltpu.make_async_copy(
                            f_hbm.at[pl.ds(0, BATCH)], bufs[b], sems[b]
                        ).wait()
                        pltpu.sync_copy(bufs[b], acc.at[dst_v.at[idx]],
                                        add=True)

                        @pl.when(idx + NBUF < CB)
                        def _start_next():
                            gstart(idx + NBUF, b)
                    return _

                with jax.named_scope("ph_pipe"):
                    for b in range(NBUF):
                        gstart(b, b)
                    lax.fori_loop(0, CB // NBUF, pipe_body, None)

            plsc.subcore_barrier()
            with jax.named_scope("ph_out"):
                @pl.when(sid < NS - 1)
                def _out_full():
                    pltpu.sync_copy(acc.at[pl.ds(sid * SLAB, SLAB)],
                                    out_hbm.at[pl.ds(sid * SLAB, SLAB)])

                @pl.when(sid == NS - 1)
                def _out_last():
                    pltpu.sync_copy(
                        acc.at[pl.ds((NS - 1) * SLAB, SLAB_LAST)],
                        out_hbm.at[pl.ds((NS - 1) * SLAB, SLAB_LAST)])

    return sc_kern(f_table, src_w, et_w, dst_w)


def kernel(feat, edge_index, etypes, weight):
    src = edge_index[0]
    dst = edge_index[1]
    pad = E_PAD - N_EDGES
    ar = jnp.arange(pad, dtype=jnp.int32)
    src_p = jnp.concatenate([src, (ar * 37) % N_NODES])
    et_p = jnp.concatenate([etypes, ar % R])
    dst_p = jnp.concatenate([dst, N_NODES + ar % (N_PAD - N_NODES)])

    f_table = _rel_matmul(feat, weight).reshape(N_NODES * R, D)
    return _sc_gather_scatter(
        f_table,
        src_p.reshape(NBT, BATCH),
        et_p.reshape(NBT, BATCH),
        dst_p.reshape(NBT, BATCH),
    )


# speedup vs baseline: 2.4663x; 1.0023x over previous
import functools

import jax
import jax.numpy as jnp
from jax import lax
from jax.experimental import pallas as pl
from jax.experimental.pallas import tpu as pltpu
from jax.experimental.pallas import tpu_sc as plsc

N_NODES = 10000
D = 128
R = 8
N_EDGES = 160000

NC = 2
NS = 16
BATCH = 64
NB = 160
NCHUNK = 4
CB = NB // NCHUNK
NBT = NS * NB
E_PAD = NBT * BATCH
N_PAD = 10112
SLAB = N_PAD // NS
SLAB_LAST = N_NODES - 15 * SLAB
NBUF = 4


def _relmm_body(f_ref, w_ref, o_ref):
    o_ref[0] = jnp.dot(f_ref[...], w_ref[0],
                       preferred_element_type=jnp.float32)


def _rel_matmul(feat, weight):
    return pl.pallas_call(
        _relmm_body,
        grid=(R,),
        in_specs=[
            pl.BlockSpec((N_NODES, D), lambda r: (0, 0)),
            pl.BlockSpec((1, D, D), lambda r: (r, 0, 0)),
        ],
        out_specs=pl.BlockSpec((1, N_NODES, D), lambda r: (r, 0, 0)),
        out_shape=jax.ShapeDtypeStruct((R, N_NODES, D), jnp.float32),
    )(feat, weight)


def _sc_gather_scatter(f_table, src_w, et_w, dst_w):
    mesh = plsc.VectorSubcoreMesh(core_axis_name="c", subcore_axis_name="s")

    @functools.partial(
        pl.kernel,
        mesh=mesh,
        out_type=jax.ShapeDtypeStruct((N_NODES, D), jnp.float32),
        scratch_types=[
            pltpu.VMEM((CB, BATCH), jnp.int32),
            pltpu.VMEM((CB, BATCH), jnp.int32),
            pltpu.VMEM((BATCH, D), jnp.float32),
            pltpu.VMEM((BATCH, D), jnp.float32),
            pltpu.VMEM((BATCH, D), jnp.float32),
            pltpu.VMEM((BATCH, D), jnp.float32),
            pltpu.VMEM_SHARED((N_PAD, D), jnp.float32),
            pltpu.SemaphoreType.DMA,
            pltpu.SemaphoreType.DMA,
            pltpu.SemaphoreType.DMA,
            pltpu.SemaphoreType.DMA,
        ],
    )
    def sc_kern(f_hbm, src_hbm, et_hbm, dst_hbm, out_hbm,
                key_v, dst_v, rows0, rows1, rows2, rows3, acc,
                sem0, sem1, sem2, sem3):
        cid = lax.axis_index("c")
        sid = lax.axis_index("s")

        @pl.when(cid == 0)
        def _all_work():
            zero16 = jnp.zeros((16,), jnp.float32)

            def zero_body(i, _):
                for c in range(D // 16):
                    rows0[i, pl.ds(c * 16, 16)] = zero16
                return _

            lax.fori_loop(0, BATCH, zero_body, None)
            for k in range(SLAB // BATCH):
                pltpu.sync_copy(
                    rows0, acc.at[pl.ds(sid * SLAB + k * BATCH, BATCH)])
            rem = SLAB % BATCH
            if rem:
                pltpu.sync_copy(
                    rows0.at[pl.ds(0, rem)],
                    acc.at[pl.ds(sid * SLAB + (SLAB // BATCH) * BATCH,
                                 rem)])

            plsc.subcore_barrier()

            bufs = (rows0, rows1, rows2, rows3)
            sems = (sem0, sem1, sem2, sem3)

            def gstart(bidx, b):
                pltpu.async_copy(f_hbm.at[key_v.at[bidx]], bufs[b], sems[b])

            for ci in range(NCHUNK):
                bofs = sid * NB + ci * CB

                pltpu.sync_copy(et_hbm.at[pl.ds(bofs, CB)],
                                key_v.at[pl.ds(0, CB)])
                pltpu.sync_copy(src_hbm.at[pl.ds(bofs, CB)],
                                dst_v.at[pl.ds(0, CB)])

                def key_body(j, _):
                    for c in range(BATCH // 16):
                        sl = pl.ds(c * 16, 16)
                        key_v[j, sl] = key_v[j, sl] * N_NODES + dst_v[j, sl]
                    return _

                lax.fori_loop(0, CB, key_body, None)

                pltpu.sync_copy(dst_hbm.at[pl.ds(bofs, CB)],
                                dst_v.at[pl.ds(0, CB)])

                def pipe_body(j, _):
                    for b in range(NBUF):
                        idx = j * NBUF + b
                        pltpu.make_async_copy(
                            f_hbm.at[pl.ds(0, BATCH)], bufs[b], sems[b]
                        ).wait()
                        pltpu.sync_copy(bufs[b], acc.at[dst_v.at[idx]],
                                        add=True)

                        @pl.when(idx + NBUF < CB)
                        def _start_next():
                            gstart(idx + NBUF, b)
                    return _

                for b in range(NBUF):
                    gstart(b, b)
                lax.fori_loop(0, CB // NBUF, pipe_body, None)

            plsc.subcore_barrier()

            @pl.when(sid < NS - 1)
            def _out_full():
                pltpu.sync_copy(acc.at[pl.ds(sid * SLAB, SLAB)],
                                out_hbm.at[pl.ds(sid * SLAB, SLAB)])

            @pl.when(sid == NS - 1)
            def _out_last():
                pltpu.sync_copy(
                    acc.at[pl.ds((NS - 1) * SLAB, SLAB_LAST)],
                    out_hbm.at[pl.ds((NS - 1) * SLAB, SLAB_LAST)])

    return sc_kern(f_table, src_w, et_w, dst_w)


def kernel(feat, edge_index, etypes, weight):
    src = edge_index[0]
    dst = edge_index[1]
    pad = E_PAD - N_EDGES
    ar = jnp.arange(pad, dtype=jnp.int32)
    src_p = jnp.concatenate([src, (ar * 37) % N_NODES])
    et_p = jnp.concatenate([etypes, ar % R])
    dst_p = jnp.concatenate([dst, N_NODES + ar % (N_PAD - N_NODES)])

    f_table = _rel_matmul(feat, weight).reshape(N_NODES * R, D)
    return _sc_gather_scatter(
        f_table,
        src_p.reshape(NBT, BATCH),
        et_p.reshape(NBT, BATCH),
        dst_p.reshape(NBT, BATCH),
    )
